# Initial kernel scaffold; baseline (speedup 1.0000x reference)
import functools

import jax
import jax.numpy as jnp
from jax import lax
from jax.experimental import pallas as pl
from jax.experimental.pallas import tpu as pltpu
from jax.experimental.pallas import tpu_sc as plsc

_B = 16384
_D = 128
_NSUB = 10
_NNEG = 10
_NW = 32
_BPW = _B // _NW
_CHUNK = 32
_NCHUNK = _BPW // _CHUNK
_L = 16


def _sc_dots(pos_u, pos_v, neg_v_flat, wd, ut, vt):
    mesh = plsc.VectorSubcoreMesh(core_axis_name="c", subcore_axis_name="s")

    @functools.partial(
        pl.kernel,
        out_type=[
            jax.ShapeDtypeStruct((_B,), jnp.float32),
            jax.ShapeDtypeStruct((_NNEG, _B), jnp.float32),
        ],
        mesh=mesh,
        scratch_types=[
            pltpu.VMEM((_CHUNK,), jnp.int32),
            pltpu.VMEM((_CHUNK,), jnp.int32),
            pltpu.VMEM((_CHUNK * _NNEG,), jnp.int32),
            pltpu.VMEM((_CHUNK, _NSUB), jnp.int32),
            pltpu.VMEM((_NSUB, _CHUNK), jnp.int32),
            pltpu.VMEM((_NSUB * _CHUNK, _D), jnp.float32),
            pltpu.VMEM((_CHUNK, _D), jnp.float32),
            pltpu.VMEM((_CHUNK * _NNEG, _D), jnp.float32),
            pltpu.VMEM((_CHUNK,), jnp.float32),
            pltpu.VMEM((_NNEG, _CHUNK), jnp.float32),
            pltpu.SemaphoreType.DMA,
        ],
    )
    def sc_kernel(pos_u_h, pos_v_h, neg_v_h, wd_h, ut_h, vt_h,
                  posd_h, negd_h,
                  puI, pvI, ngI, wdR, subT, uR, pV, nV, posd, negd, sem):
        wid = lax.axis_index("s") * 2 + lax.axis_index("c")
        base = wid * _BPW

        def chunk_body(c, carry):
            cb = pl.multiple_of(base + c * _CHUNK, _CHUNK)
            nb = pl.multiple_of(cb * _NNEG, _CHUNK * _NNEG)
            pltpu.sync_copy(pos_u_h.at[pl.ds(cb, _CHUNK)], puI)
            pltpu.sync_copy(pos_v_h.at[pl.ds(cb, _CHUNK)], pvI)
            pltpu.sync_copy(neg_v_h.at[pl.ds(nb, _CHUNK * _NNEG)], ngI)
            handles = []
            handles.append(pltpu.async_copy(vt_h.at[pvI], pV, sem))
            for j, sz in ((0, 128), (1, 128), (2, 64)):
                handles.append(pltpu.async_copy(
                    vt_h.at[ngI.at[pl.ds(j * 128, sz)]],
                    nV.at[pl.ds(j * 128, sz)], sem))
            pltpu.async_copy(wd_h.at[puI], wdR, sem).wait()
            for g in range(_CHUNK // _L):
                rows = lax.iota(jnp.int32, _L) + g * _L
                for s in range(_NSUB):
                    col = jnp.full((_L,), s, jnp.int32)
                    subT[s, pl.ds(g * _L, _L)] = plsc.load_gather(wdR, [rows, col])
            for s in range(_NSUB):
                handles.append(pltpu.async_copy(
                    ut_h.at[subT.at[s]], uR.at[pl.ds(s * _CHUNK, _CHUNK)], sem))
            for h in handles:
                h.wait()
            for g in range(_CHUNK // _L):
                rowb = lax.iota(jnp.int32, _L) + g * _L
                negbase = rowb * _NNEG

                def dbody(dd, dcarry, rowb=rowb, negbase=negbase):
                    pacc = dcarry[0]
                    naccs = dcarry[1:]
                    dspl = jnp.full((_L,), dd, jnp.int32)
                    u = plsc.load_gather(uR, [rowb, dspl])
                    for s in range(1, _NSUB):
                        u = u + plsc.load_gather(uR, [rowb + s * _CHUNK, dspl])
                    v = plsc.load_gather(pV, [rowb, dspl])
                    out = [pacc + u * v]
                    for n in range(_NNEG):
                        nv = plsc.load_gather(nV, [negbase + n, dspl])
                        out.append(naccs[n] + nv * u)
                    return tuple(out)

                z = jnp.zeros((_L,), jnp.float32)
                res = lax.fori_loop(0, _D, dbody, (z,) * (1 + _NNEG))
                posd[pl.ds(g * _L, _L)] = res[0]
                for n in range(_NNEG):
                    negd[n, pl.ds(g * _L, _L)] = res[1 + n]
            pltpu.sync_copy(posd, posd_h.at[pl.ds(cb, _CHUNK)])
            for n in range(_NNEG):
                pltpu.sync_copy(negd.at[n], negd_h.at[n, pl.ds(cb, _CHUNK)])
            return carry

        lax.fori_loop(0, _NCHUNK, chunk_body, 0)

    return sc_kernel(pos_u, pos_v, neg_v_flat, wd, ut, vt)


def _tc_loss(posd, negd):
    pos2 = posd.reshape(_B // _D, _D)
    neg2 = negd.reshape(_NNEG * _B // _D, _D)

    def body(p_ref, n_ref, o_ref):
        p = p_ref[...]
        acc = jnp.sum(jnp.minimum(p, 0.0) - jnp.log1p(jnp.exp(-jnp.abs(p))))
        q = -n_ref[...]
        acc = acc + jnp.sum(jnp.minimum(q, 0.0) - jnp.log1p(jnp.exp(-jnp.abs(q))))
        o_ref[0, 0] = -acc / _B

    out = pl.pallas_call(
        body,
        out_shape=jax.ShapeDtypeStruct((1, 1), jnp.float32),
        out_specs=pl.BlockSpec(memory_space=pltpu.SMEM),
    )(pos2, neg2)
    return out[0, 0]


def kernel(pos_u, pos_v, neg_v, wdidx2moidx, u_table, v_table):
    posd, negd = _sc_dots(pos_u, pos_v, neg_v.reshape(-1),
                          wdidx2moidx, u_table, v_table)
    return _tc_loss(posd, negd)


# speedup vs baseline: 1.1391x; 1.1391x over previous
import functools

import jax
import jax.numpy as jnp
from jax import lax
from jax.experimental import pallas as pl
from jax.experimental.pallas import tpu as pltpu
from jax.experimental.pallas import tpu_sc as plsc

_B = 16384
_D = 128
_NSUB = 10
_NNEG = 10
_NW = 32
_BPW = _B // _NW
_CHUNK = 32
_NCHUNK = _BPW // _CHUNK
_L = 16


def _sc_dots(pos_u, pos_v, neg_v_flat, wd, ut, vt):
    mesh = plsc.VectorSubcoreMesh(core_axis_name="c", subcore_axis_name="s",
                                  num_cores=2, num_subcores=16)

    @functools.partial(
        pl.kernel,
        out_type=[
            jax.ShapeDtypeStruct((_B,), jnp.float32),
            jax.ShapeDtypeStruct((_NNEG * _B,), jnp.float32),
        ],
        mesh=mesh,
        scratch_types=[
            pltpu.VMEM((_CHUNK,), jnp.int32),
            pltpu.VMEM((_CHUNK,), jnp.int32),
            pltpu.VMEM((_CHUNK * _NNEG,), jnp.int32),
            pltpu.VMEM((_NSUB * _CHUNK,), jnp.int32),
            pltpu.VMEM((_NSUB * _CHUNK,), jnp.int32),
            pltpu.VMEM((_NSUB * _CHUNK, _D), jnp.float32),
            pltpu.VMEM((_CHUNK, _D), jnp.float32),
            pltpu.VMEM((_CHUNK * _NNEG, _D), jnp.float32),
            pltpu.VMEM((_CHUNK,), jnp.float32),
            pltpu.VMEM((_NNEG * _CHUNK,), jnp.float32),
            pltpu.SemaphoreType.DMA,
            pltpu.SemaphoreType.DMA,
        ],
        compiler_params=pltpu.CompilerParams(
            needs_layout_passes=False, use_tc_tiling_on_sc=False),
    )
    def sc_kernel(pos_u_h, pos_v_h, neg_v_h, wdf_h, ut_h, vt_h,
                  posd_h, negd_h,
                  puI, pvI, ngI, subI, subT, uR, pV, nV, posd, negd, sem, sem2):
        wid = lax.axis_index("s") * 2 + lax.axis_index("c")
        base = wid * _BPW

        def chunk_body(c, carry):
            cb = pl.multiple_of(base + c * _CHUNK, _CHUNK)
            nb = pl.multiple_of(cb * _NNEG, _CHUNK * _NNEG)
            pltpu.sync_copy(pos_u_h.at[pl.ds(cb, _CHUNK)], puI)
            pltpu.sync_copy(pos_v_h.at[pl.ds(cb, _CHUNK)], pvI)
            pltpu.sync_copy(neg_v_h.at[pl.ds(nb, _CHUNK * _NNEG)], ngI)
            handles = []
            handles.append(pltpu.async_copy(vt_h.at[pvI], pV, sem))
            for j, sz in ((0, 128), (1, 128), (2, 64)):
                handles.append(pltpu.async_copy(
                    vt_h.at[ngI.at[pl.ds(j * 128, sz)]],
                    nV.at[pl.ds(j * 128, sz)], sem))
            for g in range(_CHUNK // _L):
                pub = plsc.load_gather(puI, [lax.iota(jnp.int32, _L) + g * _L])
                pub = pub * _NSUB
                for s in range(_NSUB):
                    subI[pl.ds(s * _CHUNK + g * _L, _L)] = pub + s
            wd_handles = []
            for j, sz in ((0, 128), (1, 128), (2, 64)):
                wd_handles.append(pltpu.async_copy(
                    wdf_h.at[subI.at[pl.ds(j * 128, sz)]],
                    subT.at[pl.ds(j * 128, sz)], sem2))
            for h in wd_handles:
                h.wait()
            for s in range(_NSUB):
                handles.append(pltpu.async_copy(
                    ut_h.at[subT.at[pl.ds(s * _CHUNK, _CHUNK)]],
                    uR.at[pl.ds(s * _CHUNK, _CHUNK)], sem))
            for h in handles:
                h.wait()
            for g in range(_CHUNK // _L):
                rowb = lax.iota(jnp.int32, _L) + g * _L
                negbase = rowb * _NNEG

                def dbody(dd, dcarry, rowb=rowb, negbase=negbase):
                    pacc = dcarry[0]
                    naccs = dcarry[1:]
                    dspl = jnp.full((_L,), dd, jnp.int32)
                    u = plsc.load_gather(uR, [rowb, dspl])
                    for s in range(1, _NSUB):
                        u = u + plsc.load_gather(uR, [rowb + s * _CHUNK, dspl])
                    v = plsc.load_gather(pV, [rowb, dspl])
                    out = [pacc + u * v]
                    for n in range(_NNEG):
                        nv = plsc.load_gather(nV, [negbase + n, dspl])
                        out.append(naccs[n] + nv * u)
                    return tuple(out)

                z = jnp.zeros((_L,), jnp.float32)
                res = lax.fori_loop(0, _D, dbody, (z,) * (1 + _NNEG))
                posd[pl.ds(g * _L, _L)] = res[0]
                for n in range(_NNEG):
                    negd[pl.ds(n * _CHUNK + g * _L, _L)] = res[1 + n]
            pltpu.sync_copy(posd, posd_h.at[pl.ds(cb, _CHUNK)])
            for n in range(_NNEG):
                pltpu.sync_copy(negd.at[pl.ds(n * _CHUNK, _CHUNK)],
                                negd_h.at[pl.ds(n * _B + cb, _CHUNK)])
            return carry

        lax.fori_loop(0, _NCHUNK, chunk_body, 0)

    return sc_kernel(pos_u, pos_v, neg_v_flat, wd.reshape(-1), ut, vt)


def _tc_loss(posd, negd):
    pos2 = posd.reshape(_B // _D, _D)
    neg2 = negd.reshape(_NNEG * _B // _D, _D)

    def body(p_ref, n_ref, o_ref):
        p = p_ref[...]
        acc = jnp.sum(jnp.minimum(p, 0.0) - jnp.log1p(jnp.exp(-jnp.abs(p))))
        q = -n_ref[...]
        acc = acc + jnp.sum(jnp.minimum(q, 0.0) - jnp.log1p(jnp.exp(-jnp.abs(q))))
        o_ref[0, 0] = -acc / _B

    out = pl.pallas_call(
        body,
        out_shape=jax.ShapeDtypeStruct((1, 1), jnp.float32),
        out_specs=pl.BlockSpec(memory_space=pltpu.SMEM),
    )(pos2, neg2)
    return out[0, 0]


def kernel(pos_u, pos_v, neg_v, wdidx2moidx, u_table, v_table):
    posd, negd = _sc_dots(pos_u, pos_v, neg_v.reshape(-1),
                          wdidx2moidx, u_table, v_table)
    return _tc_loss(posd, negd)


# speedup vs baseline: 3.5120x; 3.0830x over previous
import functools

import jax
import jax.numpy as jnp
from jax import lax
from jax.experimental import pallas as pl
from jax.experimental.pallas import tpu as pltpu
from jax.experimental.pallas import tpu_sc as plsc

_B = 16384
_D = 128
_NSUB = 10
_NNEG = 10
_NW = 32
_BPW = _B // _NW
_CHUNK = 32
_NCHUNK = _BPW // _CHUNK
_L = 16


def _sc_dots(pos_u, pos_v, neg_v_flat, wd, ut, vt):
    mesh = plsc.VectorSubcoreMesh(core_axis_name="c", subcore_axis_name="s",
                                  num_cores=2, num_subcores=16)

    @functools.partial(
        pl.kernel,
        out_type=[
            jax.ShapeDtypeStruct((_B * _L,), jnp.float32),
            jax.ShapeDtypeStruct((_NNEG * _B * _L,), jnp.float32),
        ],
        mesh=mesh,
        scratch_types=[
            pltpu.VMEM((_CHUNK,), jnp.int32),
            pltpu.VMEM((_CHUNK,), jnp.int32),
            pltpu.VMEM((_CHUNK * _NNEG,), jnp.int32),
            pltpu.VMEM((_NSUB * _CHUNK,), jnp.int32),
            pltpu.VMEM((_NSUB * _CHUNK,), jnp.int32),
            pltpu.VMEM((_NSUB * _CHUNK, _D), jnp.float32),
            pltpu.VMEM((_CHUNK, _D), jnp.float32),
            pltpu.VMEM((_CHUNK * _NNEG, _D), jnp.float32),
            pltpu.VMEM((_CHUNK * _L,), jnp.float32),
            pltpu.VMEM((_NNEG * _CHUNK * _L,), jnp.float32),
            pltpu.SemaphoreType.DMA,
            pltpu.SemaphoreType.DMA,
        ],
        compiler_params=pltpu.CompilerParams(
            needs_layout_passes=False, use_tc_tiling_on_sc=False),
    )
    def sc_kernel(pos_u_h, pos_v_h, neg_v_h, wdf_h, ut_h, vt_h,
                  posd_h, negd_h,
                  puI, pvI, ngI, subI, subT, uR, pV, nV, posd, negd, sem, sem2):
        wid = lax.axis_index("s") * 2 + lax.axis_index("c")
        base = wid * _BPW

        def chunk_body(c, carry):
            cb = pl.multiple_of(base + c * _CHUNK, _CHUNK)
            nb = pl.multiple_of(cb * _NNEG, _CHUNK * _NNEG)
            pltpu.sync_copy(pos_u_h.at[pl.ds(cb, _CHUNK)], puI)
            pltpu.sync_copy(pos_v_h.at[pl.ds(cb, _CHUNK)], pvI)
            pltpu.sync_copy(neg_v_h.at[pl.ds(nb, _CHUNK * _NNEG)], ngI)
            handles = []
            handles.append(pltpu.async_copy(vt_h.at[pvI], pV, sem))
            for j, sz in ((0, 128), (1, 128), (2, 64)):
                handles.append(pltpu.async_copy(
                    vt_h.at[ngI.at[pl.ds(j * 128, sz)]],
                    nV.at[pl.ds(j * 128, sz)], sem))
            for g in range(_CHUNK // _L):
                pub = plsc.load_gather(puI, [lax.iota(jnp.int32, _L) + g * _L])
                pub = pub * _NSUB
                for s in range(_NSUB):
                    subI[pl.ds(s * _CHUNK + g * _L, _L)] = pub + s
            wd_handles = []
            for j, sz in ((0, 128), (1, 128), (2, 64)):
                wd_handles.append(pltpu.async_copy(
                    wdf_h.at[subI.at[pl.ds(j * 128, sz)]],
                    subT.at[pl.ds(j * 128, sz)], sem2))
            for h in wd_handles:
                h.wait()
            for s in range(_NSUB):
                handles.append(pltpu.async_copy(
                    ut_h.at[subT.at[pl.ds(s * _CHUNK, _CHUNK)]],
                    uR.at[pl.ds(s * _CHUNK, _CHUNK)], sem))
            for h in handles:
                h.wait()
            def ebody(b, ecarry):
                pooled = []
                for k in range(_D // _L):
                    u = uR[b, pl.ds(k * _L, _L)]
                    for s in range(1, _NSUB):
                        u = u + uR[s * _CHUNK + b, pl.ds(k * _L, _L)]
                    pooled.append(u)
                pacc = pooled[0] * pV[b, pl.ds(0, _L)]
                for k in range(1, _D // _L):
                    pacc = pacc + pooled[k] * pV[b, pl.ds(k * _L, _L)]
                posd[pl.ds(b * _L, _L)] = pacc
                for n in range(_NNEG):
                    r = b * _NNEG + n
                    nacc = pooled[0] * nV[r, pl.ds(0, _L)]
                    for k in range(1, _D // _L):
                        nacc = nacc + pooled[k] * nV[r, pl.ds(k * _L, _L)]
                    negd[pl.ds((n * _CHUNK + b) * _L, _L)] = nacc
                return ecarry

            lax.fori_loop(0, _CHUNK, ebody, 0)
            pltpu.sync_copy(posd, posd_h.at[pl.ds(cb * _L, _CHUNK * _L)])
            for n in range(_NNEG):
                pltpu.sync_copy(
                    negd.at[pl.ds(n * _CHUNK * _L, _CHUNK * _L)],
                    negd_h.at[pl.ds((n * _B + cb) * _L, _CHUNK * _L)])
            return carry

        lax.fori_loop(0, _NCHUNK, chunk_body, 0)

    return sc_kernel(pos_u, pos_v, neg_v_flat, wd.reshape(-1), ut, vt)


def _tc_loss(posd, negd):
    pos2 = posd.reshape(_B * _L // 128, 128)
    neg2 = negd.reshape(_NNEG * _B * _L // 128, 128)

    def body(p_ref, n_ref, o_ref):
        row = lax.broadcasted_iota(jnp.int32, (128, 8), 0) // _L
        col = lax.broadcasted_iota(jnp.int32, (128, 8), 1)
        S = (row == col).astype(jnp.float32)
        p = jnp.dot(p_ref[...], S)
        acc = jnp.sum(jnp.minimum(p, 0.0) - jnp.log1p(jnp.exp(-jnp.abs(p))))
        q = -jnp.dot(n_ref[...], S)
        acc = acc + jnp.sum(jnp.minimum(q, 0.0) - jnp.log1p(jnp.exp(-jnp.abs(q))))
        o_ref[0, 0] = -acc / _B

    out = pl.pallas_call(
        body,
        out_shape=jax.ShapeDtypeStruct((1, 1), jnp.float32),
        out_specs=pl.BlockSpec(memory_space=pltpu.SMEM),
    )(pos2, neg2)
    return out[0, 0]


def kernel(pos_u, pos_v, neg_v, wdidx2moidx, u_table, v_table):
    posd, negd = _sc_dots(pos_u, pos_v, neg_v.reshape(-1),
                          wdidx2moidx, u_table, v_table)
    return _tc_loss(posd, negd)


# speedup vs baseline: 3.5599x; 1.0136x over previous
import functools

import jax
import jax.numpy as jnp
from jax import lax
from jax.experimental import pallas as pl
from jax.experimental.pallas import tpu as pltpu
from jax.experimental.pallas import tpu_sc as plsc

_B = 16384
_D = 128
_NSUB = 10
_NNEG = 10
_NW = 32
_BPW = _B // _NW
_CHUNK = 16
_NCHUNK = _BPW // _CHUNK
_L = 16


def _slices(total):
    out, o = [], 0
    while o < total:
        sz = min(128, total - o)
        out.append((o, sz))
        o += sz
    return out


_NG_SLC = _slices(_CHUNK * _NNEG)
_SUB_SLC = _slices(_NSUB * _CHUNK)


def _sc_dots(pos_u, pos_v, neg_v_flat, wd, ut, vt):
    mesh = plsc.VectorSubcoreMesh(core_axis_name="c", subcore_axis_name="s",
                                  num_cores=2, num_subcores=16)

    @functools.partial(
        pl.kernel,
        out_type=[
            jax.ShapeDtypeStruct((_B * _L,), jnp.float32),
            jax.ShapeDtypeStruct((_NNEG * _B * _L,), jnp.float32),
        ],
        mesh=mesh,
        scratch_types=(
            [
                pltpu.VMEM((_CHUNK,), jnp.int32),
                pltpu.VMEM((_CHUNK,), jnp.int32),
                pltpu.VMEM((_CHUNK * _NNEG,), jnp.int32),
                pltpu.VMEM((_NSUB * _CHUNK,), jnp.int32),
                pltpu.VMEM((_NSUB * _CHUNK,), jnp.int32),
                pltpu.VMEM((_NSUB * _CHUNK, _D), jnp.float32),
                pltpu.VMEM((_CHUNK, _D), jnp.float32),
                pltpu.VMEM((_CHUNK * _NNEG, _D), jnp.float32),
                pltpu.SemaphoreType.DMA,
            ] * 2 + [
                pltpu.VMEM((_CHUNK * _L,), jnp.float32),
                pltpu.VMEM((_NNEG * _CHUNK * _L,), jnp.float32),
                pltpu.SemaphoreType.DMA,
            ]),
        compiler_params=pltpu.CompilerParams(
            needs_layout_passes=False, use_tc_tiling_on_sc=False),
    )
    def sc_kernel(pos_u_h, pos_v_h, neg_v_h, wdf_h, ut_h, vt_h,
                  posd_h, negd_h, *scr):
        slotA, slotB = scr[0:9], scr[9:18]
        posd, negd, sem2 = scr[18], scr[19], scr[20]
        wid = lax.axis_index("s") * 2 + lax.axis_index("c")
        base = wid * _BPW

        def prefetch(c, slot):
            puI, pvI, ngI, subI, subT, uR, pV, nV, sem = slot
            cb = pl.multiple_of(base + c * _CHUNK, _CHUNK)
            nb = pl.multiple_of(cb * _NNEG, _CHUNK * _NNEG)
            pltpu.sync_copy(pos_u_h.at[pl.ds(cb, _CHUNK)], puI)
            pltpu.sync_copy(pos_v_h.at[pl.ds(cb, _CHUNK)], pvI)
            pltpu.sync_copy(neg_v_h.at[pl.ds(nb, _CHUNK * _NNEG)], ngI)
            pltpu.async_copy(vt_h.at[pvI], pV, sem)
            for o, sz in _NG_SLC:
                pltpu.async_copy(
                    vt_h.at[ngI.at[pl.ds(o, sz)]],
                    nV.at[pl.ds(o, sz)], sem)
            for g in range(_CHUNK // _L):
                pub = plsc.load_gather(puI, [lax.iota(jnp.int32, _L) + g * _L])
                pub = pub * _NSUB
                for s in range(_NSUB):
                    subI[pl.ds(s * _CHUNK + g * _L, _L)] = pub + s
            wd_handles = []
            for o, sz in _SUB_SLC:
                wd_handles.append(pltpu.async_copy(
                    wdf_h.at[subI.at[pl.ds(o, sz)]],
                    subT.at[pl.ds(o, sz)], sem2))
            for h in wd_handles:
                h.wait()
            for s in range(_NSUB):
                pltpu.async_copy(
                    ut_h.at[subT.at[pl.ds(s * _CHUNK, _CHUNK)]],
                    uR.at[pl.ds(s * _CHUNK, _CHUNK)], sem)

        def drain(slot):
            puI, pvI, ngI, subI, subT, uR, pV, nV, sem = slot
            pltpu.make_async_copy(vt_h.at[pl.ds(0, _CHUNK)], pV, sem).wait()
            for o, sz in _NG_SLC:
                pltpu.make_async_copy(
                    vt_h.at[pl.ds(0, sz)], nV.at[pl.ds(o, sz)],
                    sem).wait()
            for s in range(_NSUB):
                pltpu.make_async_copy(
                    ut_h.at[pl.ds(0, _CHUNK)],
                    uR.at[pl.ds(s * _CHUNK, _CHUNK)], sem).wait()

        def compute(c, slot):
            puI, pvI, ngI, subI, subT, uR, pV, nV, sem = slot
            cb = pl.multiple_of(base + c * _CHUNK, _CHUNK)

            def ebody(b, ecarry):
                pooled = []
                for k in range(_D // _L):
                    u = uR[b, pl.ds(k * _L, _L)]
                    for s in range(1, _NSUB):
                        u = u + uR[s * _CHUNK + b, pl.ds(k * _L, _L)]
                    pooled.append(u)
                pacc = pooled[0] * pV[b, pl.ds(0, _L)]
                for k in range(1, _D // _L):
                    pacc = pacc + pooled[k] * pV[b, pl.ds(k * _L, _L)]
                posd[pl.ds(b * _L, _L)] = pacc
                for n in range(_NNEG):
                    r = b * _NNEG + n
                    nacc = pooled[0] * nV[r, pl.ds(0, _L)]
                    for k in range(1, _D // _L):
                        nacc = nacc + pooled[k] * nV[r, pl.ds(k * _L, _L)]
                    negd[pl.ds((n * _CHUNK + b) * _L, _L)] = nacc
                return ecarry

            lax.fori_loop(0, _CHUNK, ebody, 0)
            pltpu.sync_copy(posd, posd_h.at[pl.ds(cb * _L, _CHUNK * _L)])
            for n in range(_NNEG):
                pltpu.sync_copy(
                    negd.at[pl.ds(n * _CHUNK * _L, _CHUNK * _L)],
                    negd_h.at[pl.ds((n * _B + cb) * _L, _CHUNK * _L)])

        prefetch(0, slotA)

        def pair_body(g, carry):
            c0 = g * 2
            drain(slotA)
            prefetch(jnp.minimum(c0 + 1, _NCHUNK - 1), slotB)
            compute(c0, slotA)
            c1 = c0 + 1
            drain(slotB)
            prefetch(jnp.minimum(c1 + 1, _NCHUNK - 1), slotA)
            compute(c1, slotB)
            return carry

        lax.fori_loop(0, _NCHUNK // 2, pair_body, 0)
        drain(slotA)

    return sc_kernel(pos_u, pos_v, neg_v_flat, wd.reshape(-1), ut, vt)


def _tc_loss(posd, negd):
    pos2 = posd.reshape(_B * _L // 128, 128)
    neg2 = negd.reshape(_NNEG * _B * _L // 128, 128)

    def body(p_ref, n_ref, o_ref):
        row = lax.broadcasted_iota(jnp.int32, (128, 8), 0) // _L
        col = lax.broadcasted_iota(jnp.int32, (128, 8), 1)
        S = (row == col).astype(jnp.float32)
        p = jnp.dot(p_ref[...], S)
        acc = jnp.sum(jnp.minimum(p, 0.0) - jnp.log1p(jnp.exp(-jnp.abs(p))))
        q = -jnp.dot(n_ref[...], S)
        acc = acc + jnp.sum(jnp.minimum(q, 0.0) - jnp.log1p(jnp.exp(-jnp.abs(q))))
        o_ref[0, 0] = -acc / _B

    out = pl.pallas_call(
        body,
        out_shape=jax.ShapeDtypeStruct((1, 1), jnp.float32),
        out_specs=pl.BlockSpec(memory_space=pltpu.SMEM),
    )(pos2, neg2)
    return out[0, 0]


def kernel(pos_u, pos_v, neg_v, wdidx2moidx, u_table, v_table):
    posd, negd = _sc_dots(pos_u, pos_v, neg_v.reshape(-1),
                          wdidx2moidx, u_table, v_table)
    return _tc_loss(posd, negd)
